# Initial kernel scaffold; baseline (speedup 1.0000x reference)
import jax
import jax.numpy as jnp
from jax.experimental import pallas as pl


def kernel(feature):
    raise NotImplementedError("write your pallas kernel here")


# speedup vs baseline: 20.3985x; 20.3985x over previous
import jax
import jax.numpy as jnp
from jax.experimental import pallas as pl
from jax.experimental.pallas import tpu as pltpu

GROUP = 16
CG = 6
P = 6
NS = 9
H = 224
HO = 222


def _win_coords(flat):
    c, k = flat // 9, flat % 9
    return c, k // 3, k % 3


def _motif_group_kernel(x_ref, o_ref):
    o_ref[...] = jnp.zeros_like(o_ref)

    def win(flat):
        c, ki, kj = _win_coords(flat)
        return x_ref[c, ki:ki + HO, kj:kj + HO]

    for j in range(NS):
        m = [win(P * j + t) for t in range(P)]

        dist = {}
        for p in range(P):
            for q in range(p + 1, P):
                d = m[p] - m[q]
                dist[(p, q)] = jnp.sqrt(jnp.sum(d * d))

        def D(p, q):
            return dist[(p, q)] if p < q else dist[(q, p)]

        nn = []
        for p in range(P):
            best = jnp.float32(jnp.inf)
            idx = jnp.int32(0)
            for q in range(P):
                if q == p:
                    continue
                better = D(p, q) < best
                best = jnp.where(better, D(p, q), best)
                idx = jnp.where(better, jnp.int32(q), idx)
            nn.append(idx)

        u = nn[0]
        for p in range(1, P):
            u = jnp.maximum(u, nn[p])
        cnt = (nn[0] == u).astype(jnp.float32)
        for p in range(1, P):
            cnt = cnt + (nn[p] == u).astype(jnp.float32)

        sel = (u == 0).astype(jnp.float32) * m[0]
        for t in range(1, P):
            sel = sel + (u == t).astype(jnp.float32) * m[t]

        for t in range(P):
            c, ki, kj = _win_coords(P * j + t)
            term = jnp.floor((m[t] * sel) * cnt / jnp.float32(CG))
            o_ref[c, ki:ki + HO, kj:kj + HO] += term


def kernel(feature):
    x = feature[0]
    out = pl.pallas_call(
        _motif_group_kernel,
        grid=(GROUP,),
        in_specs=[pl.BlockSpec((CG, H, H), lambda g: (g, 0, 0))],
        out_specs=pl.BlockSpec((CG, H, H), lambda g: (g, 0, 0)),
        out_shape=jax.ShapeDtypeStruct((GROUP * CG, H, H), jnp.float32),
        compiler_params=pltpu.CompilerParams(
            dimension_semantics=("arbitrary",),
        ),
    )(x)
    return out[None]


# speedup vs baseline: 32.1118x; 1.5742x over previous
import jax
import jax.numpy as jnp
from jax.experimental import pallas as pl
from jax.experimental.pallas import tpu as pltpu

GROUP = 16
CG = 6
P = 6
NS = 9
H = 224
HO = 222

_PASSES = [
    [(0, 1), (0, 2), (0, 3), (0, 4), (0, 5)],
    [(1, 2), (1, 3), (1, 4), (1, 5), (2, 3)],
    [(2, 4), (2, 5), (3, 4), (3, 5), (4, 5)],
]


def _win_coords(flat):
    c, k = flat // 9, flat % 9
    return c, k // 3, k % 3


def _motif_group_kernel(x_ref, o_ref, sh_ref, oa_ref):
    o_ref[...] = jnp.zeros_like(o_ref)
    oa_ref[...] = jnp.zeros_like(oa_ref)

    for c in range(CG):
        for kj in (1, 2):
            sh_ref[(kj - 1) * CG + c, :, 0:H - kj] = x_ref[c, :, kj:H]

    def wchunk(flat, rs, nr):
        c, ki, kj = _win_coords(flat)
        if kj == 0:
            return x_ref[c, ki + rs:ki + rs + nr, 0:HO]
        return sh_ref[(kj - 1) * CG + c, ki + rs:ki + rs + nr, 0:HO]

    tail_mask = jax.lax.broadcasted_iota(jnp.int32, (8, HO), 0) >= 2

    for j in range(NS):
        dist = {}
        for pair_pass in _PASSES:
            accs = {}
            for chunk in range(28):
                rs = 8 * chunk if chunk < 27 else 8 * 27 - 2
                cache = {}
                for (p, q) in pair_pass:
                    for t in (p, q):
                        if t not in cache:
                            cache[t] = wchunk(P * j + t, rs, 8)
                    d = cache[p] - cache[q]
                    dd = d * d
                    if chunk == 27:
                        dd = jnp.where(tail_mask, dd, jnp.float32(0.0))
                    accs[(p, q)] = dd if chunk == 0 else accs[(p, q)] + dd
            for pq in pair_pass:
                dist[pq] = jnp.sqrt(jnp.sum(accs[pq]))

        def D(p, q):
            return dist[(p, q)] if p < q else dist[(q, p)]

        nn = []
        for p in range(P):
            best = jnp.float32(jnp.inf)
            idx = jnp.int32(0)
            for q in range(P):
                if q == p:
                    continue
                better = D(p, q) < best
                best = jnp.where(better, D(p, q), best)
                idx = jnp.where(better, jnp.int32(q), idx)
            nn.append(idx)
        u = nn[0]
        for p in range(1, P):
            u = jnp.maximum(u, nn[p])
        cnt = (nn[0] == u).astype(jnp.float32)
        for p in range(1, P):
            cnt = cnt + (nn[p] == u).astype(jnp.float32)
        factor = cnt * jnp.float32(1.0 / CG)
        w_hot = [(u == t).astype(jnp.float32) for t in range(P)]

        for chunk in range(28):
            rs = 8 * chunk
            nr = 8 if chunk < 27 else 6
            cache = [wchunk(P * j + t, rs, nr) for t in range(P)]
            selc = w_hot[0] * cache[0]
            for t in range(1, P):
                selc = selc + w_hot[t] * cache[t]
            for t in range(P):
                c, ki, kj = _win_coords(P * j + t)
                term = jnp.floor((cache[t] * selc) * factor)
                if kj == 0:
                    o_ref[c, ki + rs:ki + rs + nr, 0:HO] += term
                else:
                    oa_ref[(kj - 1) * CG + c,
                           ki + rs:ki + rs + nr, 0:HO] += term

    for c in range(CG):
        for kj in (1, 2):
            o_ref[c, :, kj:kj + HO] += oa_ref[(kj - 1) * CG + c, :, 0:HO]


def kernel(feature):
    x = feature[0]
    out = pl.pallas_call(
        _motif_group_kernel,
        grid=(GROUP,),
        in_specs=[pl.BlockSpec((CG, H, H), lambda g: (g, 0, 0))],
        out_specs=pl.BlockSpec((CG, H, H), lambda g: (g, 0, 0)),
        out_shape=jax.ShapeDtypeStruct((GROUP * CG, H, H), jnp.float32),
        scratch_shapes=[
            pltpu.VMEM((2 * CG, H, H), jnp.float32),
            pltpu.VMEM((2 * CG, H, H), jnp.float32),
        ],
        compiler_params=pltpu.CompilerParams(
            dimension_semantics=("arbitrary",),
        ),
    )(x)
    return out[None]
